# baseline (speedup 1.0000x reference)
import functools

import jax
import jax.numpy as jnp
from jax import lax
from jax.experimental import pallas as pl
from jax.experimental.pallas import tpu as pltpu
from jax.experimental.pallas import tpu_sc as plsc

DIM = 1024
INTER = 256
E = 64
T = 2048
TB = 256
BLK = 128
NBLK = 96
NSLOT = NBLK * BLK


def _gating_body(x_ref, gw_ref, tw_ref, idx_ref, rk_ref, cnt_ref, c0_ref, c1_ref):
    i = pl.program_id(0)

    @pl.when(i == 0)
    def _():
        c0_ref[...] = jnp.zeros((1, E), jnp.float32)
        c1_ref[...] = jnp.zeros((1, E), jnp.float32)

    xb = x_ref[...]
    gw = gw_ref[...]
    logits = lax.dot_general(xb, gw, (((1,), (1,)), ((), ())),
                             preferred_element_type=jnp.float32)
    iota_e = lax.broadcasted_iota(jnp.int32, (TB, E), 1)
    m1 = jnp.max(logits, axis=1, keepdims=True)
    i1 = jnp.min(jnp.where(logits >= m1, iota_e, E), axis=1, keepdims=True)
    O0 = iota_e == i1
    l2 = jnp.where(O0, -jnp.inf, logits)
    m2 = jnp.max(l2, axis=1, keepdims=True)
    i2 = jnp.min(jnp.where(l2 >= m2, iota_e, E), axis=1, keepdims=True)
    O1 = iota_e == i2
    r = jnp.exp(m2 - m1)
    w0 = 1.0 / (1.0 + r)
    tw_ref[:, 0:1] = w0
    tw_ref[:, 1:2] = 1.0 - w0
    idx_ref[:, 0:1] = i1
    idx_ref[:, 1:2] = i2

    O0f = O0.astype(jnp.float32)
    O1f = O1.astype(jnp.float32)
    ri = lax.broadcasted_iota(jnp.int32, (TB, TB), 0)
    ci = lax.broadcasted_iota(jnp.int32, (TB, TB), 1)
    Ls = (ci < ri).astype(jnp.float32)
    c0 = c0_ref[...]
    c1 = c1_ref[...]
    ex0 = lax.dot_general(Ls, O0f, (((1,), (0,)), ((), ())),
                          preferred_element_type=jnp.float32) + c0
    ex1 = lax.dot_general(Ls, O1f, (((1,), (0,)), ((), ())),
                          preferred_element_type=jnp.float32) + c1
    rk_ref[:, 0:1] = jnp.sum(O0f * (ex0 + ex1), axis=1, keepdims=True)
    rk_ref[:, 1:2] = jnp.sum(O1f * (ex0 + ex1 + O0f), axis=1, keepdims=True)
    c0 = c0 + jnp.sum(O0f, axis=0, keepdims=True)
    c1 = c1 + jnp.sum(O1f, axis=0, keepdims=True)
    c0_ref[...] = c0
    c1_ref[...] = c1
    cnt_ref[...] = c0 + c1


def _gating(x2, gate_w, interpret=False):
    return pl.pallas_call(
        _gating_body,
        grid=(T // TB,),
        in_specs=[
            pl.BlockSpec((TB, DIM), lambda i: (i, 0)),
            pl.BlockSpec((E, DIM), lambda i: (0, 0)),
        ],
        out_specs=(
            pl.BlockSpec((TB, 2), lambda i: (i, 0)),
            pl.BlockSpec((TB, 2), lambda i: (i, 0)),
            pl.BlockSpec((TB, 2), lambda i: (i, 0)),
            pl.BlockSpec((1, E), lambda i: (0, 0)),
        ),
        out_shape=(
            jax.ShapeDtypeStruct((T, 2), jnp.float32),
            jax.ShapeDtypeStruct((T, 2), jnp.int32),
            jax.ShapeDtypeStruct((T, 2), jnp.float32),
            jax.ShapeDtypeStruct((1, E), jnp.float32),
        ),
        scratch_shapes=[
            pltpu.VMEM((1, E), jnp.float32),
            pltpu.VMEM((1, E), jnp.float32),
        ],
        interpret=interpret,
    )(x2, gate_w)


def _meta_body(idx_ref, rk_ref, cnt_ref, dest_ref, be_ref, nb_ref):
    iota_e = lax.broadcasted_iota(jnp.int32, (T, E), 1)
    O0f = (iota_e == idx_ref[:, 0:1]).astype(jnp.float32)
    O1f = (iota_e == idx_ref[:, 1:2]).astype(jnp.float32)
    cnt = cnt_ref[...]
    nb_e = jnp.floor((cnt + (BLK - 1.0)) * (1.0 / BLK))
    ri64 = lax.broadcasted_iota(jnp.int32, (E, E), 0)
    ci64 = lax.broadcasted_iota(jnp.int32, (E, E), 1)
    Us = (ri64 < ci64).astype(jnp.float32)
    exb = lax.dot_general(nb_e, Us, (((1,), (0,)), ((), ())),
                          preferred_element_type=jnp.float32)
    basep = BLK * exb
    dest0 = jnp.sum(O0f * basep, axis=1, keepdims=True) + rk_ref[:, 0:1]
    dest1 = jnp.sum(O1f * basep, axis=1, keepdims=True) + rk_ref[:, 1:2]
    dest_ref[:, 0:1] = dest0.astype(jnp.int32)
    dest_ref[:, 1:2] = dest1.astype(jnp.int32)

    bi = lax.broadcasted_iota(jnp.int32, (NBLK + 32, E), 0).astype(jnp.float32)
    cmp = (exb <= bi).astype(jnp.float32)
    be_ref[...] = (jnp.sum(cmp, axis=1, keepdims=True) - 1.0).astype(jnp.int32)
    nb_ref[...] = jnp.sum(nb_e, axis=1, keepdims=True).astype(jnp.int32)


def _meta(idx, rk, cnt, interpret=False):
    return pl.pallas_call(
        _meta_body,
        out_shape=(
            jax.ShapeDtypeStruct((T, 2), jnp.int32),
            jax.ShapeDtypeStruct((NBLK + 32, 1), jnp.int32),
            jax.ShapeDtypeStruct((1, 1), jnp.int32),
        ),
        interpret=interpret,
    )(idx, rk, cnt)


def _make_scatter_x():
    mesh = plsc.VectorSubcoreMesh(core_axis_name="c", subcore_axis_name="s")
    tpw = T // 32

    @functools.partial(
        pl.kernel, mesh=mesh,
        out_type=jax.ShapeDtypeStruct((NSLOT, DIM), jnp.float32),
        scratch_types=[
            pltpu.VMEM((tpw,), jnp.int32),
            pltpu.VMEM((tpw,), jnp.int32),
            pltpu.VMEM((tpw, DIM), jnp.float32),
            pltpu.SemaphoreType.DMA,
        ],
    )
    def sca(x_hbm, destT_hbm, xs_hbm, idx0_v, idx1_v, rows_v, sem):
        wid = lax.axis_index("s") * 2 + lax.axis_index("c")
        t0 = wid * tpw
        pltpu.sync_copy(destT_hbm.at[0, pl.ds(t0, tpw)], idx0_v)
        pltpu.sync_copy(destT_hbm.at[1, pl.ds(t0, tpw)], idx1_v)
        pltpu.sync_copy(x_hbm.at[pl.ds(t0, tpw)], rows_v)
        pltpu.async_copy(rows_v, xs_hbm.at[idx0_v], sem).wait()
        pltpu.async_copy(rows_v, xs_hbm.at[idx1_v], sem).wait()

    return sca


def _ffn_body(be_s, nb_s, xs_ref, wu_ref, wg_ref, wd_ref, ys_ref):
    i = pl.program_id(0)

    @pl.when(i < nb_s[0])
    def _():
        xb = xs_ref[...]
        wu = wu_ref[0]
        wg = wg_ref[0]
        wd = wd_ref[0]
        up = lax.dot_general(xb, wu, (((1,), (1,)), ((), ())),
                             preferred_element_type=jnp.float32)
        gt = lax.dot_general(xb, wg, (((1,), (1,)), ((), ())),
                             preferred_element_type=jnp.float32)
        gt = 0.5 * gt * (1.0 + lax.erf(gt * 0.7071067811865476))
        h = up * gt
        ys_ref[...] = lax.dot_general(h, wd, (((1,), (1,)), ((), ())),
                                      preferred_element_type=jnp.float32)


def _ffn_grouped(xs, be, nb, w_up, w_gate, w_down, interpret=False):
    grid_spec = pltpu.PrefetchScalarGridSpec(
        num_scalar_prefetch=2,
        grid=(NBLK,),
        in_specs=[
            pl.BlockSpec((BLK, DIM),
                         lambda i, be, nb: (jnp.minimum(i, nb[0] - 1), 0)),
            pl.BlockSpec((1, INTER, DIM),
                         lambda i, be, nb: (be[jnp.minimum(i, nb[0] - 1)], 0, 0)),
            pl.BlockSpec((1, INTER, DIM),
                         lambda i, be, nb: (be[jnp.minimum(i, nb[0] - 1)], 0, 0)),
            pl.BlockSpec((1, DIM, INTER),
                         lambda i, be, nb: (be[jnp.minimum(i, nb[0] - 1)], 0, 0)),
        ],
        out_specs=pl.BlockSpec((BLK, DIM),
                               lambda i, be, nb: (jnp.minimum(i, nb[0] - 1), 0)),
    )
    return pl.pallas_call(
        _ffn_body,
        grid_spec=grid_spec,
        out_shape=jax.ShapeDtypeStruct((NSLOT, DIM), jnp.float32),
        interpret=interpret,
    )(be, nb, xs, w_up, w_gate, w_down)


def _make_gather_y():
    mesh = plsc.VectorSubcoreMesh(core_axis_name="c", subcore_axis_name="s")
    tpw = T // 32

    @functools.partial(
        pl.kernel, mesh=mesh,
        out_type=jax.ShapeDtypeStruct((2, T, DIM), jnp.float32),
        scratch_types=[
            pltpu.VMEM((tpw,), jnp.int32),
            pltpu.VMEM((tpw, DIM), jnp.float32),
            pltpu.SemaphoreType.DMA,
        ],
    )
    def scb(ys_hbm, destT_hbm, yp_hbm, idx_v, rows_v, sem):
        wid = lax.axis_index("s") * 2 + lax.axis_index("c")
        t0 = wid * tpw
        pltpu.sync_copy(destT_hbm.at[0, pl.ds(t0, tpw)], idx_v)
        pltpu.async_copy(ys_hbm.at[idx_v], rows_v, sem).wait()
        pltpu.sync_copy(rows_v, yp_hbm.at[0, pl.ds(t0, tpw)])
        pltpu.sync_copy(destT_hbm.at[1, pl.ds(t0, tpw)], idx_v)
        pltpu.async_copy(ys_hbm.at[idx_v], rows_v, sem).wait()
        pltpu.sync_copy(rows_v, yp_hbm.at[1, pl.ds(t0, tpw)])

    return scb


def _shared_body(x_ref, swu_ref, swg_ref, swd_ref, sh_ref):
    xb = x_ref[...]
    up = lax.dot_general(xb, swu_ref[...], (((1,), (1,)), ((), ())),
                         preferred_element_type=jnp.float32)
    gt = lax.dot_general(xb, swg_ref[...], (((1,), (1,)), ((), ())),
                         preferred_element_type=jnp.float32)
    gt = 0.5 * gt * (1.0 + lax.erf(gt * 0.7071067811865476))
    sh_ref[...] = lax.dot_general(up * gt, swd_ref[...], (((1,), (1,)), ((), ())),
                                  preferred_element_type=jnp.float32)


def _shared_half(xh, sw_up, sw_gate, sw_down, interpret=False):
    TH = T // 2
    return pl.pallas_call(
        _shared_body,
        grid=(TH // TB,),
        in_specs=[
            pl.BlockSpec((TB, DIM), lambda i: (i, 0)),
            pl.BlockSpec((INTER, DIM), lambda i: (0, 0)),
            pl.BlockSpec((INTER, DIM), lambda i: (0, 0)),
            pl.BlockSpec((DIM, INTER), lambda i: (0, 0)),
        ],
        out_specs=pl.BlockSpec((TB, DIM), lambda i: (i, 0)),
        out_shape=jax.ShapeDtypeStruct((TH, DIM), jnp.float32),
        interpret=interpret,
    )(xh, sw_up, sw_gate, sw_down)


def _combine_body(sh1_ref, sh2_ref, yp_ref, tw_ref, out_ref):
    i = pl.program_id(0)
    HB = (T // 2) // TB
    sh = jnp.where(i < HB, sh1_ref[...], sh2_ref[...])
    w0 = tw_ref[:, 0:1]
    w1 = tw_ref[:, 1:2]
    out_ref[...] = sh + w0 * yp_ref[0] + w1 * yp_ref[1]


def _combine(sh1, sh2, yp, tw, interpret=False):
    HB = (T // 2) // TB
    return pl.pallas_call(
        _combine_body,
        grid=(T // TB,),
        in_specs=[
            pl.BlockSpec((TB, DIM), lambda i: (jnp.minimum(i, HB - 1), 0)),
            pl.BlockSpec((TB, DIM), lambda i: (jnp.maximum(i - HB, 0), 0)),
            pl.BlockSpec((2, TB, DIM), lambda i: (0, i, 0)),
            pl.BlockSpec((TB, 2), lambda i: (i, 0)),
        ],
        out_specs=pl.BlockSpec((TB, DIM), lambda i: (i, 0)),
        out_shape=jax.ShapeDtypeStruct((T, DIM), jnp.float32),
        interpret=interpret,
    )(sh1, sh2, yp, tw)


def kernel(x, gate_w, w_up, w_gate, w_down, sw_up, sw_gate, sw_down):
    orig_shape = x.shape
    x2 = x.reshape(-1, orig_shape[-1])
    tw, idx, rk, cnt = _gating(x2, gate_w)
    dest, be, nb = _meta(idx, rk, cnt)
    destT = dest.T
    be_flat = be.reshape(-1)[:NBLK]
    nb_flat = nb.reshape(-1)
    xs = _make_scatter_x()(x2, destT)
    sh1 = _shared_half(x2[: T // 2], sw_up, sw_gate, sw_down)
    ys = _ffn_grouped(xs, be_flat, nb_flat, w_up, w_gate, w_down)
    yp = _make_gather_y()(ys, destT)
    sh2 = _shared_half(x2[T // 2:], sw_up, sw_gate, sw_down)
    out = _combine(sh1, sh2, yp, tw)
    return out.reshape(orig_shape)


# speedup vs baseline: 1.2232x; 1.2232x over previous
import functools

import jax
import jax.numpy as jnp
from jax import lax
from jax.experimental import pallas as pl
from jax.experimental.pallas import tpu as pltpu
from jax.experimental.pallas import tpu_sc as plsc

DIM = 1024
INTER = 256
E = 64
T = 2048
BLK = 128
NBLK = 96
NSLOT = NBLK * BLK


def _pack16(v16):
    h = v16.shape[1] // 2
    au = lax.bitcast_convert_type(v16[:, :h], jnp.uint16).astype(jnp.uint32)
    bu = lax.bitcast_convert_type(v16[:, h:], jnp.uint16).astype(jnp.uint32)
    return au | (bu << 16)


def _unpack16(w):
    lo = lax.bitcast_convert_type((w & 0xFFFF).astype(jnp.uint16), jnp.bfloat16)
    hi = lax.bitcast_convert_type((w >> 16).astype(jnp.uint16), jnp.bfloat16)
    return jnp.concatenate([lo, hi], axis=1).astype(jnp.float32)


def _gating_meta_body(x_ref, gw_ref, tw_ref, dest_ref, be_ref, nb_ref, xb_ref):
    xx = x_ref[...]
    gw = gw_ref[...]
    xb_ref[...] = _pack16(xx.astype(jnp.bfloat16))
    logits = lax.dot_general(xx, gw, (((1,), (1,)), ((), ())),
                             preferred_element_type=jnp.float32)
    iota_e = lax.broadcasted_iota(jnp.int32, (T, E), 1)
    m1 = jnp.max(logits, axis=1, keepdims=True)
    i1 = jnp.min(jnp.where(logits >= m1, iota_e, E), axis=1, keepdims=True)
    O0 = iota_e == i1
    l2 = jnp.where(O0, -jnp.inf, logits)
    m2 = jnp.max(l2, axis=1, keepdims=True)
    i2 = jnp.min(jnp.where(l2 >= m2, iota_e, E), axis=1, keepdims=True)
    O1 = iota_e == i2
    r = jnp.exp(m2 - m1)
    w0 = 1.0 / (1.0 + r)
    tw_ref[:, 0:1] = w0
    tw_ref[:, 1:2] = 1.0 - w0

    O0f = O0.astype(jnp.float32)
    O1f = O1.astype(jnp.float32)
    ri = lax.broadcasted_iota(jnp.int32, (256, 256), 0)
    ci = lax.broadcasted_iota(jnp.int32, (256, 256), 1)
    Ls = (ci < ri).astype(jnp.float32)
    c0 = jnp.zeros((1, E), jnp.float32)
    c1 = jnp.zeros((1, E), jnp.float32)
    ex0, ex1 = [], []
    for b in range(T // 256):
        O0b = O0f[b * 256:(b + 1) * 256]
        O1b = O1f[b * 256:(b + 1) * 256]
        ex0.append(lax.dot_general(Ls, O0b, (((1,), (0,)), ((), ())),
                                   preferred_element_type=jnp.float32) + c0)
        ex1.append(lax.dot_general(Ls, O1b, (((1,), (0,)), ((), ())),
                                   preferred_element_type=jnp.float32) + c1)
        c0 = c0 + jnp.sum(O0b, axis=0, keepdims=True)
        c1 = c1 + jnp.sum(O1b, axis=0, keepdims=True)
    C0ex = jnp.concatenate(ex0, axis=0)
    C1ex = jnp.concatenate(ex1, axis=0)
    rank0 = C0ex + C1ex
    rank1 = C0ex + C1ex + O0f

    cnt = c0 + c1
    nb_e = jnp.floor((cnt + (BLK - 1.0)) * (1.0 / BLK))
    ri64 = lax.broadcasted_iota(jnp.int32, (E, E), 0)
    ci64 = lax.broadcasted_iota(jnp.int32, (E, E), 1)
    Us = (ri64 < ci64).astype(jnp.float32)
    exb = lax.dot_general(nb_e, Us, (((1,), (0,)), ((), ())),
                          preferred_element_type=jnp.float32)
    basep = BLK * exb
    dest0 = jnp.sum(O0f * (basep + rank0), axis=1, keepdims=True)
    dest1 = jnp.sum(O1f * (basep + rank1), axis=1, keepdims=True)
    dest_ref[:, 0:1] = dest0.astype(jnp.int32)
    dest_ref[:, 1:2] = dest1.astype(jnp.int32)

    bi = lax.broadcasted_iota(jnp.int32, (NBLK + 32, E), 0).astype(jnp.float32)
    cmp = (exb <= bi).astype(jnp.float32)
    be_ref[...] = (jnp.sum(cmp, axis=1, keepdims=True) - 1.0).astype(jnp.int32)
    nb_ref[...] = jnp.sum(nb_e, axis=1, keepdims=True).astype(jnp.int32)


def _gating_meta(x2, gate_w, interpret=False):
    return pl.pallas_call(
        _gating_meta_body,
        out_shape=(
            jax.ShapeDtypeStruct((T, 2), jnp.float32),
            jax.ShapeDtypeStruct((T, 2), jnp.int32),
            jax.ShapeDtypeStruct((NBLK + 32, 1), jnp.int32),
            jax.ShapeDtypeStruct((1, 1), jnp.int32),
            jax.ShapeDtypeStruct((T, DIM // 2), jnp.uint32),
        ),
        interpret=interpret,
    )(x2, gate_w)


def _make_scatter_x():
    mesh = plsc.VectorSubcoreMesh(core_axis_name="c", subcore_axis_name="s")
    tpw = T // 32

    @functools.partial(
        pl.kernel, mesh=mesh,
        out_type=jax.ShapeDtypeStruct((NSLOT, DIM // 2), jnp.uint32),
        scratch_types=[
            pltpu.VMEM((tpw,), jnp.int32),
            pltpu.VMEM((tpw,), jnp.int32),
            pltpu.VMEM((tpw, DIM // 2), jnp.uint32),
            pltpu.SemaphoreType.DMA,
        ],
    )
    def sca(xb16_hbm, destT_hbm, xs_hbm, idx0_v, idx1_v, rows_v, sem):
        wid = lax.axis_index("s") * 2 + lax.axis_index("c")
        t0 = wid * tpw
        pltpu.sync_copy(destT_hbm.at[0, pl.ds(t0, tpw)], idx0_v)
        pltpu.sync_copy(destT_hbm.at[1, pl.ds(t0, tpw)], idx1_v)
        pltpu.sync_copy(xb16_hbm.at[pl.ds(t0, tpw)], rows_v)
        pltpu.async_copy(rows_v, xs_hbm.at[idx0_v], sem).wait()
        pltpu.async_copy(rows_v, xs_hbm.at[idx1_v], sem).wait()

    return sca


def _ffn_body(be_s, nb_s, xs_ref, wu_ref, wg_ref, wd_ref, ys_ref):
    i = pl.program_id(0)

    @pl.when(i < nb_s[0])
    def _():
        xb = _unpack16(xs_ref[...])
        wu = wu_ref[0]
        wg = wg_ref[0]
        wd = wd_ref[0]
        up = lax.dot_general(xb, wu, (((1,), (1,)), ((), ())),
                             preferred_element_type=jnp.float32)
        gt = lax.dot_general(xb, wg, (((1,), (1,)), ((), ())),
                             preferred_element_type=jnp.float32)
        gt = 0.5 * gt * (1.0 + lax.erf(gt * 0.7071067811865476))
        h = up * gt
        y = lax.dot_general(h, wd, (((1,), (1,)), ((), ())),
                            preferred_element_type=jnp.float32)
        ys_ref[...] = _pack16(y.astype(jnp.bfloat16))


def _ffn_grouped(xs, be, nb, w_up, w_gate, w_down, interpret=False):
    grid_spec = pltpu.PrefetchScalarGridSpec(
        num_scalar_prefetch=2,
        grid=(NBLK,),
        in_specs=[
            pl.BlockSpec((BLK, DIM // 2),
                         lambda i, be, nb: (jnp.minimum(i, nb[0] - 1), 0)),
            pl.BlockSpec((1, INTER, DIM),
                         lambda i, be, nb: (be[jnp.minimum(i, nb[0] - 1)], 0, 0)),
            pl.BlockSpec((1, INTER, DIM),
                         lambda i, be, nb: (be[jnp.minimum(i, nb[0] - 1)], 0, 0)),
            pl.BlockSpec((1, DIM, INTER),
                         lambda i, be, nb: (be[jnp.minimum(i, nb[0] - 1)], 0, 0)),
        ],
        out_specs=pl.BlockSpec((BLK, DIM // 2),
                               lambda i, be, nb: (jnp.minimum(i, nb[0] - 1), 0)),
    )
    return pl.pallas_call(
        _ffn_body,
        grid_spec=grid_spec,
        out_shape=jax.ShapeDtypeStruct((NSLOT, DIM // 2), jnp.uint32),
        interpret=interpret,
    )(be, nb, xs, w_up, w_gate, w_down)


def _make_gather_y():
    mesh = plsc.VectorSubcoreMesh(core_axis_name="c", subcore_axis_name="s")
    tpw = T // 32

    @functools.partial(
        pl.kernel, mesh=mesh,
        out_type=jax.ShapeDtypeStruct((2, T, DIM // 2), jnp.uint32),
        scratch_types=[
            pltpu.VMEM((tpw,), jnp.int32),
            pltpu.VMEM((tpw, DIM // 2), jnp.uint32),
            pltpu.SemaphoreType.DMA,
        ],
    )
    def scb(ys_hbm, destT_hbm, yp_hbm, idx_v, rows_v, sem):
        wid = lax.axis_index("s") * 2 + lax.axis_index("c")
        t0 = wid * tpw
        pltpu.sync_copy(destT_hbm.at[0, pl.ds(t0, tpw)], idx_v)
        pltpu.async_copy(ys_hbm.at[idx_v], rows_v, sem).wait()
        pltpu.sync_copy(rows_v, yp_hbm.at[0, pl.ds(t0, tpw)])
        pltpu.sync_copy(destT_hbm.at[1, pl.ds(t0, tpw)], idx_v)
        pltpu.async_copy(ys_hbm.at[idx_v], rows_v, sem).wait()
        pltpu.sync_copy(rows_v, yp_hbm.at[1, pl.ds(t0, tpw)])

    return scb


def _combine_body(x_ref, swu_ref, swg_ref, swd_ref, yp_ref, tw_ref, out_ref):
    xb = x_ref[...]
    up = lax.dot_general(xb, swu_ref[...], (((1,), (1,)), ((), ())),
                         preferred_element_type=jnp.float32)
    gt = lax.dot_general(xb, swg_ref[...], (((1,), (1,)), ((), ())),
                         preferred_element_type=jnp.float32)
    gt = 0.5 * gt * (1.0 + lax.erf(gt * 0.7071067811865476))
    sh = lax.dot_general(up * gt, swd_ref[...], (((1,), (1,)), ((), ())),
                         preferred_element_type=jnp.float32)
    w0 = tw_ref[:, 0:1]
    w1 = tw_ref[:, 1:2]
    out_ref[...] = (sh + w0 * _unpack16(yp_ref[0])
                    + w1 * _unpack16(yp_ref[1]))


def _combine(x2, sw_up, sw_gate, sw_down, yp, tw, interpret=False):
    TB = 256
    return pl.pallas_call(
        _combine_body,
        grid=(T // TB,),
        in_specs=[
            pl.BlockSpec((TB, DIM), lambda i: (i, 0)),
            pl.BlockSpec((INTER, DIM), lambda i: (0, 0)),
            pl.BlockSpec((INTER, DIM), lambda i: (0, 0)),
            pl.BlockSpec((DIM, INTER), lambda i: (0, 0)),
            pl.BlockSpec((2, TB, DIM // 2), lambda i: (0, i, 0)),
            pl.BlockSpec((TB, 2), lambda i: (i, 0)),
        ],
        out_specs=pl.BlockSpec((TB, DIM), lambda i: (i, 0)),
        out_shape=jax.ShapeDtypeStruct((T, DIM), jnp.float32),
        interpret=interpret,
    )(x2, sw_up, sw_gate, sw_down, yp, tw)


def kernel(x, gate_w, w_up, w_gate, w_down, sw_up, sw_gate, sw_down):
    orig_shape = x.shape
    x2 = x.reshape(-1, orig_shape[-1])
    tw, dest, be, nb, xb16 = _gating_meta(x2, gate_w)
    destT = dest.T
    be_flat = be.reshape(-1)[:NBLK]
    nb_flat = nb.reshape(-1)
    xs = _make_scatter_x()(xb16, destT)
    ys = _ffn_grouped(xs, be_flat, nb_flat, w_up, w_gate, w_down)
    yp = _make_gather_y()(ys, destT)
    out = _combine(x2, sw_up, sw_gate, sw_down, yp, tw)
    return out.reshape(orig_shape)
